# baseline (speedup 1.0000x reference)
import functools

import jax
import jax.numpy as jnp
from jax import lax
from jax.experimental import pallas as pl

_PAD = -1
_ND = 128
_LO, _HI = -1.0, 1.0


def _rank_body(zc, yc, xc, zr, yr, xr, rank_ref, *, ic_size, jc_size):
    jc = pl.program_id(2)

    @pl.when(jc == 0)
    def _init():
        rank_ref[...] = jnp.zeros_like(rank_ref)

    zi, yi, xi = zc[0], yc[0], xc[0]
    zj, yj, xj = zr[0], yr[0], xr[0]
    ic = pl.program_id(1)
    ig = ic * ic_size + lax.broadcasted_iota(jnp.int32, (ic_size, 1), 0)
    jg = jc * jc_size + lax.broadcasted_iota(jnp.int32, (1, jc_size), 1)
    tri = jg < ig
    lz, ez = zj < zi, zj == zi
    ly, ey = yj < yi, yj == yi
    lx, ex = xj < xi, xj == xi
    less = lz | (ez & (ly | (ey & (lx | (ex & tri)))))
    rank_ref[...] += jnp.sum(less.astype(jnp.int32), axis=1)[None, :, None]


def _ranks(z, y, x, ic_size, jc_size):
    b, n = z.shape
    col = lambda a: a[:, :, None]
    row = lambda a: a[:, None, :]
    cspec = pl.BlockSpec((1, ic_size, 1), lambda bb, i, j: (bb, i, 0))
    rspec = pl.BlockSpec((1, 1, jc_size), lambda bb, i, j: (bb, 0, j))
    out = pl.pallas_call(
        functools.partial(_rank_body, ic_size=ic_size, jc_size=jc_size),
        grid=(b, n // ic_size, n // jc_size),
        in_specs=[cspec, cspec, cspec, rspec, rspec, rspec],
        out_specs=pl.BlockSpec((1, ic_size, 1), lambda bb, i, j: (bb, i, 0)),
        out_shape=jax.ShapeDtypeStruct((b, n, 1), jnp.int32),
    )(col(z), col(y), col(x), row(z), row(y), row(x))
    return out[:, :, 0]


def _face_body(fc_ref, tok_ref):
    c = fc_ref[0]
    v = [[c[3 * k + d] for d in range(3)] for k in range(3)]

    def before(a, b):
        lz, ez = v[a][2] < v[b][2], v[a][2] == v[b][2]
        ly, ey = v[a][1] < v[b][1], v[a][1] == v[b][1]
        lx = v[a][0] < v[b][0]
        ex = v[a][0] == v[b][0]
        return lz | (ez & (ly | (ey & (lx | ex))))

    b01, b02, b12 = before(0, 1), before(0, 2), before(1, 2)
    one = jnp.ones_like(c[0], jnp.int32)
    zero = jnp.zeros_like(one)
    pos = [
        jnp.where(b01, zero, one) + jnp.where(b02, zero, one),
        jnp.where(b01, one, zero) + jnp.where(b12, zero, one),
        jnp.where(b02, one, zero) + jnp.where(b12, one, zero),
    ]
    rows = []
    for j in range(3):
        for d in range(3):
            acc = jnp.zeros_like(c[0])
            for k in range(3):
                acc = acc + jnp.where(pos[k] == j, v[k][d], 0.0)
            rows.append(acc)
    s = jnp.stack(rows)
    t = (s - _LO) / (_HI - _LO) * _ND - 0.5
    tok_ref[0] = jnp.clip(jnp.round(t).astype(jnp.int32), 0, _ND - 1)


def _face_stage(fc9):
    b, _, nf = fc9.shape
    return pl.pallas_call(
        _face_body,
        grid=(b,),
        in_specs=[pl.BlockSpec((1, 9, nf), lambda i: (i, 0, 0))],
        out_specs=pl.BlockSpec((1, 9, nf), lambda i: (i, 0, 0)),
        out_shape=jax.ShapeDtypeStruct((b, 9, nf), jnp.int32),
    )(fc9)


def _recon_body(tok_ref, recon_ref):
    d = tok_ref[...]
    recon_ref[...] = (d.astype(jnp.float32) + 0.5) / _ND * (_HI - _LO) + _LO


def kernel(vertices, faces):
    b, nv, _ = vertices.shape
    nf = faces.shape[1]
    mn = vertices.min(axis=0)
    mx = vertices.max(axis=0)
    center = (mn + mx) / 2.0
    longest = (mx - mn).max()
    v = (vertices - center) / longest

    rank_v = _ranks(v[:, :, 2], v[:, :, 1], v[:, :, 0], 2048, 2048)
    vs = jnp.zeros_like(v)
    vs = jax.vmap(lambda dst, r, src: dst.at[r].set(src))(vs, rank_v, v)

    fc = jax.vmap(lambda vv, ff: vv[ff])(vs, faces)
    fc9 = fc.reshape(b, nf, 9).transpose(0, 2, 1)
    tok = _face_stage(fc9)

    cent = fc.mean(axis=2)
    rank_f = _ranks(cent[:, :, 2], cent[:, :, 1], cent[:, :, 0], 2048, 2048)
    tokT = tok.transpose(0, 2, 1)
    out_tok = jnp.zeros_like(tokT)
    out_tok = jax.vmap(lambda dst, r, src: dst.at[r].set(src))(out_tok, rank_f, tokT)

    ids = out_tok.reshape(b, nf * 9)
    rows = nf * 9 // 128
    recon = pl.pallas_call(
        _recon_body,
        grid=(b,),
        in_specs=[pl.BlockSpec((1, rows, 128), lambda i: (i, 0, 0))],
        out_specs=pl.BlockSpec((1, rows, 128), lambda i: (i, 0, 0)),
        out_shape=jax.ShapeDtypeStruct((b, rows, 128), jnp.float32),
    )(ids.reshape(b, rows, 128)).reshape(b, nf, 3, 3)

    codes = ids.reshape(b, nf, 3, 3)
    ph = jnp.full((b, 1), _PAD, jnp.int32)
    input_ids_full = jnp.concatenate([ph, ids, ph], axis=1)
    phf = ph.astype(jnp.float32)
    ones = jnp.ones((b, nf * 9), jnp.float32)
    attn_full = jnp.concatenate([phf, ones, phf], axis=1)
    return (input_ids_full, attn_full, codes, codes, recon)


# speedup vs baseline: 1.2966x; 1.2966x over previous
import functools

import jax
import jax.numpy as jnp
from jax import lax
from jax.experimental import pallas as pl
from jax.experimental.pallas import tpu as pltpu

_PAD = -1
_ND = 128
_LO, _HI = -1.0, 1.0


def _rank_body(zc, yc, xc, zr, yr, xr, rank_ref, *, ic_size, jc_size):
    jc = pl.program_id(2)

    @pl.when(jc == 0)
    def _init():
        rank_ref[...] = jnp.zeros_like(rank_ref)

    zi, yi, xi = zc[0], yc[0], xc[0]
    zj, yj, xj = zr[0], yr[0], xr[0]
    lz, ez = zj < zi, zj == zi
    rank_ref[...] += jnp.sum(lz.astype(jnp.int32), axis=1)[None, :, None]

    @pl.when(jnp.any(ez))
    def _ties():
        ic = pl.program_id(1)
        ig = ic * ic_size + lax.broadcasted_iota(jnp.int32, (ic_size, 1), 0)
        jg = jc * jc_size + lax.broadcasted_iota(jnp.int32, (1, jc_size), 1)
        tri = jg < ig
        ly, ey = yj < yi, yj == yi
        lx, ex = xj < xi, xj == xi
        tie = ez & (ly | (ey & (lx | (ex & tri))))
        rank_ref[...] += jnp.sum(tie.astype(jnp.int32), axis=1)[None, :, None]


def _ranks(z, y, x, ic_size, jc_size):
    b, n = z.shape
    col = lambda a: a[:, :, None]
    row = lambda a: a[:, None, :]
    cspec = pl.BlockSpec((1, ic_size, 1), lambda bb, i, j: (bb, i, 0))
    rspec = pl.BlockSpec((1, 1, jc_size), lambda bb, i, j: (bb, 0, j))
    out = pl.pallas_call(
        functools.partial(_rank_body, ic_size=ic_size, jc_size=jc_size),
        grid=(b, n // ic_size, n // jc_size),
        in_specs=[cspec, cspec, cspec, rspec, rspec, rspec],
        out_specs=pl.BlockSpec((1, ic_size, 1), lambda bb, i, j: (bb, i, 0)),
        out_shape=jax.ShapeDtypeStruct((b, n, 1), jnp.int32),
        compiler_params=pltpu.CompilerParams(
            dimension_semantics=("parallel", "parallel", "arbitrary")),
    )(col(z), col(y), col(x), row(z), row(y), row(x))
    return out[:, :, 0]


def _face_body(fc_ref, tok_ref):
    c = fc_ref[0]
    v = [[c[3 * k + d] for d in range(3)] for k in range(3)]

    def before(a, b):
        lz, ez = v[a][2] < v[b][2], v[a][2] == v[b][2]
        ly, ey = v[a][1] < v[b][1], v[a][1] == v[b][1]
        lx = v[a][0] < v[b][0]
        ex = v[a][0] == v[b][0]
        return lz | (ez & (ly | (ey & (lx | ex))))

    b01, b02, b12 = before(0, 1), before(0, 2), before(1, 2)
    one = jnp.ones_like(c[0], jnp.int32)
    zero = jnp.zeros_like(one)
    pos = [
        jnp.where(b01, zero, one) + jnp.where(b02, zero, one),
        jnp.where(b01, one, zero) + jnp.where(b12, zero, one),
        jnp.where(b02, one, zero) + jnp.where(b12, one, zero),
    ]
    rows = []
    for j in range(3):
        for d in range(3):
            acc = jnp.zeros_like(c[0])
            for k in range(3):
                acc = acc + jnp.where(pos[k] == j, v[k][d], 0.0)
            rows.append(acc)
    s = jnp.stack(rows)
    t = (s - _LO) / (_HI - _LO) * _ND - 0.5
    tok_ref[0] = jnp.clip(jnp.round(t).astype(jnp.int32), 0, _ND - 1)


def _face_stage(fc9):
    b, _, nf = fc9.shape
    return pl.pallas_call(
        _face_body,
        grid=(b,),
        in_specs=[pl.BlockSpec((1, 9, nf), lambda i: (i, 0, 0))],
        out_specs=pl.BlockSpec((1, 9, nf), lambda i: (i, 0, 0)),
        out_shape=jax.ShapeDtypeStruct((b, 9, nf), jnp.int32),
    )(fc9)


def _recon_body(tok_ref, recon_ref):
    d = tok_ref[...]
    recon_ref[...] = (d.astype(jnp.float32) + 0.5) / _ND * (_HI - _LO) + _LO


def kernel(vertices, faces):
    b, nv, _ = vertices.shape
    nf = faces.shape[1]
    mn = vertices.min(axis=0)
    mx = vertices.max(axis=0)
    center = (mn + mx) / 2.0
    longest = (mx - mn).max()
    v = (vertices - center) / longest

    rank_v = _ranks(v[:, :, 2], v[:, :, 1], v[:, :, 0], 2048, 2048)
    vs = jnp.zeros_like(v)
    vs = jax.vmap(lambda dst, r, src: dst.at[r].set(src))(vs, rank_v, v)

    fc = jax.vmap(lambda vv, ff: vv[ff])(vs, faces)
    fc9 = fc.reshape(b, nf, 9).transpose(0, 2, 1)
    tok = _face_stage(fc9)

    cent = fc.mean(axis=2)
    rank_f = _ranks(cent[:, :, 2], cent[:, :, 1], cent[:, :, 0], 2048, 2048)
    tokT = tok.transpose(0, 2, 1)
    out_tok = jnp.zeros_like(tokT)
    out_tok = jax.vmap(lambda dst, r, src: dst.at[r].set(src))(out_tok, rank_f, tokT)

    ids = out_tok.reshape(b, nf * 9)
    rows = nf * 9 // 128
    recon = pl.pallas_call(
        _recon_body,
        grid=(b,),
        in_specs=[pl.BlockSpec((1, rows, 128), lambda i: (i, 0, 0))],
        out_specs=pl.BlockSpec((1, rows, 128), lambda i: (i, 0, 0)),
        out_shape=jax.ShapeDtypeStruct((b, rows, 128), jnp.float32),
    )(ids.reshape(b, rows, 128)).reshape(b, nf, 3, 3)

    codes = ids.reshape(b, nf, 3, 3)
    ph = jnp.full((b, 1), _PAD, jnp.int32)
    input_ids_full = jnp.concatenate([ph, ids, ph], axis=1)
    phf = ph.astype(jnp.float32)
    ones = jnp.ones((b, nf * 9), jnp.float32)
    attn_full = jnp.concatenate([phf, ones, phf], axis=1)
    return (input_ids_full, attn_full, codes, codes, recon)
